# baseline (speedup 1.0000x reference)
import jax
import jax.numpy as jnp
from jax.experimental import pallas as pl
from jax.experimental.pallas import tpu as pltpu


def _tile(n, target):
    t = min(target, n)
    while n % t:
        t -= 1
    return t


def _conv0_matrix(w0, W):
    Cout = w0.shape[-1]
    w_in = jnp.arange(W)[:, None]
    w_out = jnp.arange(W)[None, :]
    dx = w_in - w_out + 1
    valid = (dx >= 0) & (dx < 3)
    tap = w0[:, :, 0, :]
    A = tap[:, jnp.clip(dx, 0, 2), :]
    A = jnp.where(valid[None, :, :, None], A, 0.0)
    return A.reshape(3 * W, W * Cout)


def _c0_kernel(x_ref, a_ref, b_ref, o_ref):
    T, Hp2, W = x_ref.shape
    H = Hp2 - 2
    rows3 = jnp.concatenate(
        [x_ref[:, 0:H, :], x_ref[:, 1:H + 1, :], x_ref[:, 2:H + 2, :]],
        axis=-1).reshape(T * H, 3 * W)
    acc = jnp.dot(rows3, a_ref[...], preferred_element_type=jnp.float32)
    acc = jnp.maximum(acc + b_ref[...], 0.0)
    o_ref[...] = acc.astype(jnp.bfloat16).reshape(T, H, o_ref.shape[-1])


def _fused_kernel(x_ref, w1, b1, w2, b2, w3, b3, w4, b4, w5, b5,
                  f1, g1, f2, g2, o_ref, pad1, pad2, pad3, pad4, pad5):
    T = x_ref.shape[0]

    def conv(pad_ref, x, w_ref, b_ref, pool):
        _, Hp2, Wp2, Cin = pad_ref.shape
        H, W = Hp2 - 2, Wp2 - 2
        Cout = w_ref.shape[-1]
        pad_ref[...] = jnp.zeros_like(pad_ref)
        pad_ref[:, 1:H + 1, 1:W + 1, :] = x
        taps = [pad_ref[:, dy:dy + H, dx:dx + W, :]
                for dy in range(3) for dx in range(3)]
        col = jnp.concatenate(taps, axis=-1).reshape(T * H * W, 9 * Cin)
        acc = jnp.dot(col, w_ref[...], preferred_element_type=jnp.float32)
        acc = jnp.maximum(acc + b_ref[...], 0.0)
        if pool:
            Hp, Wp = H // 2, W // 2
            y = jnp.max(acc.reshape(T * H * Wp, 2, Cout), axis=1)
            y = y.reshape(T * Hp, 2, Wp, Cout)
            y = jnp.maximum(y[:, 0], y[:, 1])
            return y.reshape(T, Hp, Wp, Cout).astype(jnp.bfloat16)
        return acc.reshape(T, H, W, Cout).astype(jnp.bfloat16)

    y = conv(pad1, x_ref[...], w1, b1, pool=True)
    y = conv(pad2, y, w2, b2, pool=False)
    y = conv(pad3, y, w3, b3, pool=True)
    y = conv(pad4, y, w4, b4, pool=False)
    y = conv(pad5, y, w5, b5, pool=True)

    hacc = jnp.zeros((T, f1.shape[-1]), jnp.float32)
    for hh in range(4):
        for ww in range(4):
            hacc = hacc + jnp.dot(y[:, hh, ww, :], f1[hh * 4 + ww],
                                  preferred_element_type=jnp.float32)
    h = jnp.maximum(hacc + g1[...], 0.0).astype(jnp.bfloat16)
    out = jnp.dot(h, f2[...], preferred_element_type=jnp.float32) + g2[...]
    o_ref[...] = out


def kernel(x_nchw, conv0_w, conv0_b, conv1_w, conv1_b, conv2_w, conv2_b,
           conv3_w, conv3_b, conv4_w, conv4_b, conv5_w, conv5_b,
           fc1_w, fc1_b, fc2_w, fc2_b):
    N = x_nchw.shape[0]
    H = W = 32
    C0 = conv0_w.shape[-1]
    bf16 = jnp.bfloat16

    xp = jnp.pad(x_nchw[:, 0].astype(bf16), ((0, 0), (1, 1), (0, 0)))
    A = _conv0_matrix(conv0_w, W).astype(bf16)
    b0 = jnp.tile(conv0_b, W).reshape(1, W * C0)
    T0 = _tile(N, 64)
    y0 = pl.pallas_call(
        _c0_kernel,
        out_shape=jax.ShapeDtypeStruct((N, H, W * C0), bf16),
        grid=(N // T0,),
        in_specs=[
            pl.BlockSpec((T0, H + 2, W), lambda n: (n, 0, 0)),
            pl.BlockSpec((3 * W, W * C0), lambda n: (0, 0)),
            pl.BlockSpec((1, W * C0), lambda n: (0, 0)),
        ],
        out_specs=pl.BlockSpec((T0, H, W * C0), lambda n: (n, 0, 0)),
        compiler_params=pltpu.CompilerParams(
            dimension_semantics=("parallel",),
            vmem_limit_bytes=64 * 1024 * 1024),
    )(xp, A, b0)
    y0 = y0.reshape(N, H, W, C0)

    ws = []
    for w, b in ((conv1_w, conv1_b), (conv2_w, conv2_b), (conv3_w, conv3_b),
                 (conv4_w, conv4_b), (conv5_w, conv5_b)):
        cin, cout = w.shape[2], w.shape[3]
        ws.append((w.reshape(9 * cin, cout).astype(bf16), b.reshape(1, cout)))
    f1 = fc1_w.reshape(16, 128, fc1_w.shape[1]).astype(bf16)
    f2 = fc2_w.astype(bf16)
    K = fc2_w.shape[1]

    T = _tile(N, 32)
    const2 = lambda n: (0, 0)
    in_specs = [pl.BlockSpec((T, H, W, C0), lambda n: (n, 0, 0, 0))]
    args = [y0]
    for wm, bm in ws:
        in_specs.append(pl.BlockSpec(wm.shape, const2))
        in_specs.append(pl.BlockSpec(bm.shape, const2))
        args.append(wm)
        args.append(bm)
    in_specs += [
        pl.BlockSpec(f1.shape, lambda n: (0, 0, 0)),
        pl.BlockSpec(fc1_b.shape, const2),
        pl.BlockSpec(f2.shape, const2),
        pl.BlockSpec(fc2_b.shape, const2),
    ]
    args += [f1, fc1_b, f2, fc2_b]

    out = pl.pallas_call(
        _fused_kernel,
        out_shape=jax.ShapeDtypeStruct((N, K), jnp.float32),
        grid=(N // T,),
        in_specs=in_specs,
        out_specs=pl.BlockSpec((T, K), lambda n: (n, 0)),
        scratch_shapes=[
            pltpu.VMEM((T, 34, 34, 32), bf16),
            pltpu.VMEM((T, 18, 18, 32), bf16),
            pltpu.VMEM((T, 18, 18, 64), bf16),
            pltpu.VMEM((T, 10, 10, 64), bf16),
            pltpu.VMEM((T, 10, 10, 128), bf16),
        ],
        compiler_params=pltpu.CompilerParams(
            dimension_semantics=("parallel",),
            vmem_limit_bytes=64 * 1024 * 1024),
    )(*args)
    return out


# speedup vs baseline: 3.9458x; 3.9458x over previous
import jax
import jax.numpy as jnp
from jax.experimental import pallas as pl
from jax.experimental.pallas import tpu as pltpu


def _tile(n, target):
    t = min(target, n)
    while n % t:
        t -= 1
    return t


def _conv0_matrix(w0, W):
    Cout = w0.shape[-1]
    w_in = jnp.arange(W)[:, None]
    w_out = jnp.arange(W)[None, :]
    dx = w_in - w_out + 1
    valid = (dx >= 0) & (dx < 3)
    tap = w0[:, :, 0, :]
    A = tap[:, jnp.clip(dx, 0, 2), :]
    A = jnp.where(valid[None, :, :, None], A, 0.0)
    return A.reshape(3 * W, W * Cout)


def _blocked_matrix(w):
    Cin, Cout = w.shape[2], w.shape[3]
    p = jnp.array([0, 1, 2, 3, -1, 4])
    wb = jnp.arange(4)
    dx = p[:, None] - wb[None, :] + 1
    valid = (dx >= 0) & (dx < 3)
    A = w[:, jnp.clip(dx, 0, 2)]
    A = jnp.where(valid[None, :, :, None, None], A, 0.0)
    A = A.transpose(0, 1, 3, 2, 4)
    return A.reshape(3, 6 * Cin, 4 * Cout)


def _c0_kernel(x_ref, a_ref, b_ref, o_ref):
    T, Hp2, W = x_ref.shape
    H = Hp2 - 2
    rows3 = jnp.concatenate(
        [x_ref[:, 0:H, :], x_ref[:, 1:H + 1, :], x_ref[:, 2:H + 2, :]],
        axis=-1).reshape(T * H, 3 * W)
    acc = jnp.dot(rows3, a_ref[...], preferred_element_type=jnp.float32)
    acc = jnp.maximum(acc + b_ref[...], 0.0)
    o_ref[...] = acc.astype(jnp.bfloat16).reshape(T, H, o_ref.shape[-1])


def _fused_kernel(x_ref, w1, b1, w2, b2, w3, b3, w4, b4, w5, b5,
                  f1, g1, f2, g2, o_ref):
    T = x_ref.shape[0]
    bf16 = jnp.bfloat16

    def conv(xb, w_ref, b_ref, pool):
        _, H, Q, C4 = xb.shape
        Cin = C4 // 4
        C4o = w_ref.shape[-1]
        Cout = C4o // 4
        zrow = jnp.zeros((T, 1, Q, C4), bf16)
        src = jnp.concatenate([zrow, xb, zrow], axis=1)
        zq = jnp.zeros((T, H + 2, 1, Cin), bf16)
        pt = jnp.concatenate([zq, src[:, :, 0:Q - 1, 3 * Cin:4 * Cin]], axis=2)
        nh = jnp.concatenate([src[:, :, 1:Q, 0:Cin], zq], axis=2)
        win = jnp.concatenate([src, pt, nh], axis=-1)
        acc = None
        for dy in range(3):
            lhs = win[:, dy:dy + H].reshape(T * H * Q, 6 * Cin)
            part = jnp.dot(lhs, w_ref[dy], preferred_element_type=jnp.float32)
            acc = part if acc is None else acc + part
        acc = jnp.maximum(acc + b_ref[...], 0.0)
        y = acc.reshape(T, H, Q, C4o)
        if not pool:
            return y.astype(bf16)
        H2, Q2 = H // 2, Q // 2
        x5 = y.reshape(T, H2, 2, Q, C4o)
        v = jnp.maximum(x5[:, :, 0], x5[:, :, 1])
        x6 = v.reshape(T, H2, Q2, 2, C4o)
        qe, qo = x6[:, :, :, 0, :], x6[:, :, :, 1, :]
        pieces = []
        for z in (qe, qo):
            pieces.append(jnp.maximum(z[..., 0:Cout], z[..., Cout:2 * Cout]))
            pieces.append(jnp.maximum(z[..., 2 * Cout:3 * Cout],
                                      z[..., 3 * Cout:4 * Cout]))
        return jnp.concatenate(pieces, axis=-1).astype(bf16)

    y = conv(x_ref[...], w1, b1, pool=True)
    y = conv(y, w2, b2, pool=False)
    y = conv(y, w3, b3, pool=True)
    y = conv(y, w4, b4, pool=False)
    y = conv(y, w5, b5, pool=True)

    hacc = None
    for r in range(4):
        part = jnp.dot(y[:, r, 0, :], f1[r], preferred_element_type=jnp.float32)
        hacc = part if hacc is None else hacc + part
    h = jnp.maximum(hacc + g1[...], 0.0).astype(bf16)
    o_ref[...] = jnp.dot(h, f2[...], preferred_element_type=jnp.float32) + g2[...]


def kernel(x_nchw, conv0_w, conv0_b, conv1_w, conv1_b, conv2_w, conv2_b,
           conv3_w, conv3_b, conv4_w, conv4_b, conv5_w, conv5_b,
           fc1_w, fc1_b, fc2_w, fc2_b):
    N = x_nchw.shape[0]
    H = W = 32
    C0 = conv0_w.shape[-1]
    bf16 = jnp.bfloat16

    xp = jnp.pad(x_nchw[:, 0].astype(bf16), ((0, 0), (1, 1), (0, 0)))
    A = _conv0_matrix(conv0_w, W).astype(bf16)
    b0 = jnp.tile(conv0_b, W).reshape(1, W * C0)
    T0 = _tile(N, 64)
    y0 = pl.pallas_call(
        _c0_kernel,
        out_shape=jax.ShapeDtypeStruct((N, H, W * C0), bf16),
        grid=(N // T0,),
        in_specs=[
            pl.BlockSpec((T0, H + 2, W), lambda n: (n, 0, 0)),
            pl.BlockSpec((3 * W, W * C0), lambda n: (0, 0)),
            pl.BlockSpec((1, W * C0), lambda n: (0, 0)),
        ],
        out_specs=pl.BlockSpec((T0, H, W * C0), lambda n: (n, 0, 0)),
        compiler_params=pltpu.CompilerParams(
            dimension_semantics=("parallel",),
            vmem_limit_bytes=64 * 1024 * 1024),
    )(xp, A, b0)
    y0 = y0.reshape(N, H, W // 4, 4 * C0)

    ws = []
    for w, b in ((conv1_w, conv1_b), (conv2_w, conv2_b), (conv3_w, conv3_b),
                 (conv4_w, conv4_b), (conv5_w, conv5_b)):
        cout = w.shape[3]
        ws.append((_blocked_matrix(w).astype(bf16),
                   jnp.tile(b, 4).reshape(1, 4 * cout)))
    f1 = fc1_w.reshape(4, 512, fc1_w.shape[1]).astype(bf16)
    f2 = fc2_w.astype(bf16)
    K = fc2_w.shape[1]

    T = _tile(N, 32)
    const2 = lambda n: (0, 0)
    const3 = lambda n: (0, 0, 0)
    in_specs = [pl.BlockSpec((T, H, W // 4, 4 * C0), lambda n: (n, 0, 0, 0))]
    args = [y0]
    for wm, bm in ws:
        in_specs.append(pl.BlockSpec(wm.shape, const3))
        in_specs.append(pl.BlockSpec(bm.shape, const2))
        args.append(wm)
        args.append(bm)
    in_specs += [
        pl.BlockSpec(f1.shape, const3),
        pl.BlockSpec(fc1_b.shape, const2),
        pl.BlockSpec(f2.shape, const2),
        pl.BlockSpec(fc2_b.shape, const2),
    ]
    args += [f1, fc1_b, f2, fc2_b]

    out = pl.pallas_call(
        _fused_kernel,
        out_shape=jax.ShapeDtypeStruct((N, K), jnp.float32),
        grid=(N // T,),
        in_specs=in_specs,
        out_specs=pl.BlockSpec((T, K), lambda n: (n, 0)),
        compiler_params=pltpu.CompilerParams(
            dimension_semantics=("parallel",),
            vmem_limit_bytes=64 * 1024 * 1024),
    )(*args)
    return out


# speedup vs baseline: 3.9736x; 1.0070x over previous
import jax
import jax.numpy as jnp
from jax.experimental import pallas as pl
from jax.experimental.pallas import tpu as pltpu


def _tile(n, target):
    t = min(target, n)
    while n % t:
        t -= 1
    return t


def _conv0_matrix(w0, W):
    Cout = w0.shape[-1]
    w_in = jnp.arange(W)[:, None]
    w_out = jnp.arange(W)[None, :]
    dx = w_in - w_out + 1
    valid = (dx >= 0) & (dx < 3)
    tap = w0[:, :, 0, :]
    A = tap[:, jnp.clip(dx, 0, 2), :]
    A = jnp.where(valid[None, :, :, None], A, 0.0)
    return A.reshape(3 * W, W * Cout)


def _blocked_matrix(w):
    Cin, Cout = w.shape[2], w.shape[3]
    p = jnp.array([0, 1, 2, 3, -1, 4])
    wb = jnp.arange(4)
    dx = p[:, None] - wb[None, :] + 1
    valid = (dx >= 0) & (dx < 3)
    A = w[:, jnp.clip(dx, 0, 2)]
    A = jnp.where(valid[None, :, :, None, None], A, 0.0)
    A = A.transpose(1, 3, 0, 2, 4)
    return A.reshape(6 * Cin, 12 * Cout)


def _fused_kernel(x_ref, a0, c0b, w1, b1, w2, b2, w3, b3, w4, b4, w5, b5,
                  f1, g1, f2, g2, o_ref):
    T, H0, W0 = x_ref.shape
    bf16 = jnp.bfloat16

    zr = jnp.zeros((T, 1, W0), bf16)
    xpad = jnp.concatenate([zr, x_ref[...], zr], axis=1)
    rows3 = jnp.concatenate(
        [xpad[:, 0:H0, :], xpad[:, 1:H0 + 1, :], xpad[:, 2:H0 + 2, :]],
        axis=-1).reshape(T * H0, 3 * W0)
    pieces = []
    for q in range(W0 // 4):
        d = jnp.dot(rows3, a0[:, q * 128:(q + 1) * 128],
                    preferred_element_type=jnp.float32)
        d = jnp.maximum(d + c0b[...], 0.0).astype(bf16)
        pieces.append(d.reshape(T, H0, 1, 128))
    y = jnp.concatenate(pieces, axis=2)

    def conv(xb, w_ref, b_ref, pool):
        _, H, Q, C4 = xb.shape
        Cin = C4 // 4
        C4o = w_ref.shape[-1] // 3
        Cout = C4o // 4
        zrow = jnp.zeros((T, 1, Q, C4), bf16)
        src = jnp.concatenate([zrow, xb, zrow], axis=1)
        zq = jnp.zeros((T, H + 2, 1, Cin), bf16)
        pt = jnp.concatenate([zq, src[:, :, 0:Q - 1, 3 * Cin:4 * Cin]], axis=2)
        nh = jnp.concatenate([src[:, :, 1:Q, 0:Cin], zq], axis=2)
        win = jnp.concatenate([src, pt, nh], axis=-1)
        part = jnp.dot(win.reshape(T * (H + 2) * Q, 6 * Cin), w_ref[...],
                       preferred_element_type=jnp.float32)
        p4 = part.reshape(T, H + 2, Q, 3 * C4o)
        acc = (p4[:, 0:H, :, 0:C4o]
               + p4[:, 1:H + 1, :, C4o:2 * C4o]
               + p4[:, 2:H + 2, :, 2 * C4o:3 * C4o])
        acc = jnp.maximum(acc + b_ref[...].reshape(1, 1, 1, C4o), 0.0)
        if not pool:
            return acc.astype(bf16)
        H2, Q2 = H // 2, Q // 2
        x5 = acc.reshape(T, H2, 2, Q, C4o)
        v = jnp.maximum(x5[:, :, 0], x5[:, :, 1])
        x6 = v.reshape(T, H2, Q2, 2, C4o)
        qe, qo = x6[:, :, :, 0, :], x6[:, :, :, 1, :]
        out = []
        for z in (qe, qo):
            out.append(jnp.maximum(z[..., 0:Cout], z[..., Cout:2 * Cout]))
            out.append(jnp.maximum(z[..., 2 * Cout:3 * Cout],
                                   z[..., 3 * Cout:4 * Cout]))
        return jnp.concatenate(out, axis=-1).astype(bf16)

    y = conv(y, w1, b1, pool=True)
    y = conv(y, w2, b2, pool=False)
    y = conv(y, w3, b3, pool=True)
    y = conv(y, w4, b4, pool=False)
    y = conv(y, w5, b5, pool=True)

    hacc = None
    for r in range(4):
        part = jnp.dot(y[:, r, 0, :], f1[r], preferred_element_type=jnp.float32)
        hacc = part if hacc is None else hacc + part
    h = jnp.maximum(hacc + g1[...], 0.0).astype(bf16)
    o_ref[...] = jnp.dot(h, f2[...], preferred_element_type=jnp.float32) + g2[...]


def kernel(x_nchw, conv0_w, conv0_b, conv1_w, conv1_b, conv2_w, conv2_b,
           conv3_w, conv3_b, conv4_w, conv4_b, conv5_w, conv5_b,
           fc1_w, fc1_b, fc2_w, fc2_b):
    N = x_nchw.shape[0]
    H = W = 32
    bf16 = jnp.bfloat16

    xb = x_nchw[:, 0].astype(bf16)
    A = _conv0_matrix(conv0_w, W).astype(bf16)
    c0b = jnp.tile(conv0_b, 4).reshape(1, 128)

    ws = []
    for w, b in ((conv1_w, conv1_b), (conv2_w, conv2_b), (conv3_w, conv3_b),
                 (conv4_w, conv4_b), (conv5_w, conv5_b)):
        cout = w.shape[3]
        ws.append((_blocked_matrix(w).astype(bf16),
                   jnp.tile(b, 4).reshape(1, 4 * cout)))
    f1 = fc1_w.reshape(4, 512, fc1_w.shape[1]).astype(bf16)
    f2 = fc2_w.astype(bf16)
    K = fc2_w.shape[1]

    T = _tile(N, 32)
    const2 = lambda n: (0, 0)
    const3 = lambda n: (0, 0, 0)
    in_specs = [
        pl.BlockSpec((T, H, W), lambda n: (n, 0, 0)),
        pl.BlockSpec(A.shape, const2),
        pl.BlockSpec(c0b.shape, const2),
    ]
    args = [xb, A, c0b]
    for wm, bm in ws:
        in_specs.append(pl.BlockSpec(wm.shape, const2))
        in_specs.append(pl.BlockSpec(bm.shape, const2))
        args.append(wm)
        args.append(bm)
    in_specs += [
        pl.BlockSpec(f1.shape, const3),
        pl.BlockSpec(fc1_b.shape, const2),
        pl.BlockSpec(f2.shape, const2),
        pl.BlockSpec(fc2_b.shape, const2),
    ]
    args += [f1, fc1_b, f2, fc2_b]

    out = pl.pallas_call(
        _fused_kernel,
        out_shape=jax.ShapeDtypeStruct((N, K), jnp.float32),
        grid=(N // T,),
        in_specs=in_specs,
        out_specs=pl.BlockSpec((T, K), lambda n: (n, 0)),
        compiler_params=pltpu.CompilerParams(
            dimension_semantics=("parallel",),
            vmem_limit_bytes=64 * 1024 * 1024),
    )(*args)
    return out


# speedup vs baseline: 4.0108x; 1.0094x over previous
import jax
import jax.numpy as jnp
from jax.experimental import pallas as pl
from jax.experimental.pallas import tpu as pltpu


def _tile(n, target):
    t = min(target, n)
    while n % t:
        t -= 1
    return t


def _conv0_matrix(w0, W):
    Cout = w0.shape[-1]
    w_in = jnp.arange(W)[:, None]
    w_out = jnp.arange(W)[None, :]
    dx = w_in - w_out + 1
    valid = (dx >= 0) & (dx < 3)
    tap = w0[:, :, 0, :]
    A = tap[:, jnp.clip(dx, 0, 2), :]
    A = jnp.where(valid[None, :, :, None], A, 0.0)
    return A.reshape(3 * W, W * Cout)


def _blocked_matrix(w):
    Cin, Cout = w.shape[2], w.shape[3]
    p = jnp.array([0, 1, 2, 3, -1, 4])
    wb = jnp.arange(4)
    dx = p[:, None] - wb[None, :] + 1
    valid = (dx >= 0) & (dx < 3)
    A = w[:, jnp.clip(dx, 0, 2)]
    A = jnp.where(valid[None, :, :, None, None], A, 0.0)
    A = A.transpose(1, 3, 0, 2, 4)
    return A.reshape(6 * Cin, 12 * Cout)


def _fused_kernel(x_ref, a0, c0b, w1, b1, w2, b2, w3, b3, w4, b4, w5, b5,
                  f1, g1, f2, g2, o_ref):
    T, H0, W0 = x_ref.shape
    bf16 = jnp.bfloat16

    zr = jnp.zeros((T, 1, W0), bf16)
    xpad = jnp.concatenate([zr, x_ref[...], zr], axis=1)
    rows3 = jnp.concatenate(
        [xpad[:, 0:H0, :], xpad[:, 1:H0 + 1, :], xpad[:, 2:H0 + 2, :]],
        axis=-1).reshape(T * H0, 3 * W0)
    pieces = []
    for q in range(0, W0 // 4, 2):
        d = jnp.dot(rows3, a0[:, q * 128:(q + 2) * 128],
                    preferred_element_type=jnp.float32)
        d = jnp.maximum(d + c0b[...], 0.0).astype(bf16)
        pieces.append(d[:, 0:128].reshape(T, H0, 1, 128))
        pieces.append(d[:, 128:256].reshape(T, H0, 1, 128))
    y = jnp.concatenate(pieces, axis=2)

    def conv(xb, w_ref, b_ref, pool):
        _, H, Q, C4 = xb.shape
        Cin = C4 // 4
        C4o = w_ref.shape[-1] // 3
        Cout = C4o // 4
        zrow = jnp.zeros((T, 1, Q, C4), bf16)
        src = jnp.concatenate([zrow, xb, zrow], axis=1)
        zq = jnp.zeros((T, H + 2, 1, Cin), bf16)
        pt = jnp.concatenate([zq, src[:, :, 0:Q - 1, 3 * Cin:4 * Cin]], axis=2)
        nh = jnp.concatenate([src[:, :, 1:Q, 0:Cin], zq], axis=2)
        win = jnp.concatenate([src, pt, nh], axis=-1)
        part = jnp.dot(win.reshape(T * (H + 2) * Q, 6 * Cin), w_ref[...],
                       preferred_element_type=jnp.float32)
        p4 = part.reshape(T, H + 2, Q, 3 * C4o)
        acc = (p4[:, 0:H, :, 0:C4o]
               + p4[:, 1:H + 1, :, C4o:2 * C4o]
               + p4[:, 2:H + 2, :, 2 * C4o:3 * C4o])
        acc = jnp.maximum(acc + b_ref[...].reshape(1, 1, 1, C4o), 0.0)
        if not pool:
            return acc.astype(bf16)
        H2, Q2 = H // 2, Q // 2
        x5 = acc.reshape(T, H2, 2, Q, C4o)
        v = jnp.maximum(x5[:, :, 0], x5[:, :, 1])
        x6 = v.reshape(T, H2, Q2, 2, C4o)
        qe, qo = x6[:, :, :, 0, :], x6[:, :, :, 1, :]
        out = []
        for z in (qe, qo):
            out.append(jnp.maximum(z[..., 0:Cout], z[..., Cout:2 * Cout]))
            out.append(jnp.maximum(z[..., 2 * Cout:3 * Cout],
                                   z[..., 3 * Cout:4 * Cout]))
        return jnp.concatenate(out, axis=-1).astype(bf16)

    y = conv(y, w1, b1, pool=True)
    y = conv(y, w2, b2, pool=False)
    y = conv(y, w3, b3, pool=True)
    y = conv(y, w4, b4, pool=False)
    y = conv(y, w5, b5, pool=True)

    hacc = None
    for r in range(4):
        part = jnp.dot(y[:, r, 0, :], f1[r], preferred_element_type=jnp.float32)
        hacc = part if hacc is None else hacc + part
    h = jnp.maximum(hacc + g1[...], 0.0).astype(bf16)
    o_ref[...] = jnp.dot(h, f2[...], preferred_element_type=jnp.float32) + g2[...]


def kernel(x_nchw, conv0_w, conv0_b, conv1_w, conv1_b, conv2_w, conv2_b,
           conv3_w, conv3_b, conv4_w, conv4_b, conv5_w, conv5_b,
           fc1_w, fc1_b, fc2_w, fc2_b):
    N = x_nchw.shape[0]
    H = W = 32
    bf16 = jnp.bfloat16

    xb = x_nchw[:, 0].astype(bf16)
    A = _conv0_matrix(conv0_w, W).astype(bf16)
    c0b = jnp.tile(conv0_b, 8).reshape(1, 256)

    ws = []
    for w, b in ((conv1_w, conv1_b), (conv2_w, conv2_b), (conv3_w, conv3_b),
                 (conv4_w, conv4_b), (conv5_w, conv5_b)):
        cout = w.shape[3]
        ws.append((_blocked_matrix(w).astype(bf16),
                   jnp.tile(b, 4).reshape(1, 4 * cout)))
    f1 = fc1_w.reshape(4, 512, fc1_w.shape[1]).astype(bf16)
    f2 = fc2_w.astype(bf16)
    K = fc2_w.shape[1]

    T = _tile(N, 32)
    const2 = lambda n: (0, 0)
    const3 = lambda n: (0, 0, 0)
    in_specs = [
        pl.BlockSpec((T, H, W), lambda n: (n, 0, 0)),
        pl.BlockSpec(A.shape, const2),
        pl.BlockSpec(c0b.shape, const2),
    ]
    args = [xb, A, c0b]
    for wm, bm in ws:
        in_specs.append(pl.BlockSpec(wm.shape, const2))
        in_specs.append(pl.BlockSpec(bm.shape, const2))
        args.append(wm)
        args.append(bm)
    in_specs += [
        pl.BlockSpec(f1.shape, const3),
        pl.BlockSpec(fc1_b.shape, const2),
        pl.BlockSpec(f2.shape, const2),
        pl.BlockSpec(fc2_b.shape, const2),
    ]
    args += [f1, fc1_b, f2, fc2_b]

    out = pl.pallas_call(
        _fused_kernel,
        out_shape=jax.ShapeDtypeStruct((N, K), jnp.float32),
        grid=(N // T,),
        in_specs=in_specs,
        out_specs=pl.BlockSpec((T, K), lambda n: (n, 0)),
        compiler_params=pltpu.CompilerParams(
            dimension_semantics=("parallel",),
            vmem_limit_bytes=64 * 1024 * 1024),
    )(*args)
    return out


# speedup vs baseline: 4.7393x; 1.1816x over previous
import jax
import jax.numpy as jnp
from jax.experimental import pallas as pl
from jax.experimental.pallas import tpu as pltpu


def _tile(n, target):
    t = min(target, n)
    while n % t:
        t -= 1
    return t


def _conv0_matrix(w0, W):
    Cout = w0.shape[-1]
    w_in = jnp.arange(W)[:, None]
    w_out = jnp.arange(W)[None, :]
    dx = w_in - w_out + 1
    valid = (dx >= 0) & (dx < 3)
    tap = w0[:, :, 0, :]
    A = tap[:, jnp.clip(dx, 0, 2), :]
    A = jnp.where(valid[None, :, :, None], A, 0.0)
    return A.reshape(3 * W, W * Cout)


def _blocked_matrix(w, b):
    Cin, Cout = w.shape[2], w.shape[3]
    p = jnp.array(list(range(b)) + [-1, b])
    wb = jnp.arange(b)
    dx = p[:, None] - wb[None, :] + 1
    valid = (dx >= 0) & (dx < 3)
    A = w[:, jnp.clip(dx, 0, 2)]
    A = jnp.where(valid[None, :, :, None, None], A, 0.0)
    A = A.transpose(1, 3, 0, 2, 4)
    return A.reshape((b + 2) * Cin, 3 * b * Cout)


def _fused_kernel(x_ref, a0, c0b, w1, b1, w2, b2, w3, b3, w4, b4, w5, b5,
                  f1, g1, f2, g2, o_ref):
    T, H0, W0 = x_ref.shape
    bf16 = jnp.bfloat16

    zr = jnp.zeros((T, 1, W0), bf16)
    xpad = jnp.concatenate([zr, x_ref[...], zr], axis=1)
    rows3 = jnp.concatenate(
        [xpad[:, 0:H0, :], xpad[:, 1:H0 + 1, :], xpad[:, 2:H0 + 2, :]],
        axis=-1).reshape(T * H0, 3 * W0)
    pieces = []
    for q in range(0, W0 // 4, 2):
        d = jnp.dot(rows3, a0[:, q * 128:(q + 2) * 128],
                    preferred_element_type=jnp.float32)
        d = jnp.maximum(d + c0b[...], 0.0).astype(bf16)
        pieces.append(d[:, 0:128].reshape(T, H0, 1, 128))
        pieces.append(d[:, 128:256].reshape(T, H0, 1, 128))
    y = jnp.concatenate(pieces, axis=2)

    def conv(xb, w_ref, b_ref, bw, pool_ob=0):
        _, H, Q, C4 = xb.shape
        Cin = C4 // bw
        C4o = w_ref.shape[-1] // 3
        Cout = C4o // bw
        zrow = jnp.zeros((T, 1, Q, C4), bf16)
        src = jnp.concatenate([zrow, xb, zrow], axis=1)
        zq = jnp.zeros((T, H + 2, 1, Cin), bf16)
        pt = jnp.concatenate(
            [zq, src[:, :, 0:Q - 1, (bw - 1) * Cin:bw * Cin]], axis=2)
        nh = jnp.concatenate([src[:, :, 1:Q, 0:Cin], zq], axis=2)
        win = jnp.concatenate([src, pt, nh], axis=-1)
        part = jnp.dot(win.reshape(T * (H + 2) * Q, (bw + 2) * Cin), w_ref[...],
                       preferred_element_type=jnp.float32)
        p4 = part.reshape(T, H + 2, Q, 3 * C4o)
        acc = (p4[:, 0:H, :, 0:C4o]
               + p4[:, 1:H + 1, :, C4o:2 * C4o]
               + p4[:, 2:H + 2, :, 2 * C4o:3 * C4o])
        acc = jnp.maximum(acc + b_ref[...].reshape(1, 1, 1, C4o), 0.0)
        if not pool_ob:
            return acc.astype(bf16)
        H2 = H // 2
        x5 = acc.reshape(T, H2, 2, Q, C4o)
        v = jnp.maximum(x5[:, :, 0], x5[:, :, 1])
        hpieces = [jnp.maximum(v[..., (2 * j) * Cout:(2 * j + 1) * Cout],
                               v[..., (2 * j + 1) * Cout:(2 * j + 2) * Cout])
                   for j in range(bw // 2)]
        if pool_ob == bw // 2:
            return jnp.concatenate(hpieces, axis=-1).astype(bf16) \
                if len(hpieces) > 1 else hpieces[0].astype(bf16)
        vv = jnp.concatenate(hpieces, axis=-1)
        x6 = vv.reshape(T, H2, Q // 2, 2, (bw // 2) * Cout)
        return jnp.concatenate([x6[:, :, :, 0, :], x6[:, :, :, 1, :]],
                               axis=-1).astype(bf16)

    y = conv(y, w1, b1, bw=4, pool_ob=4)
    y = conv(y, w2, b2, bw=4)
    y = conv(y, w3, b3, bw=4, pool_ob=2)
    y = conv(y, w4, b4, bw=2)
    y = conv(y, w5, b5, bw=2, pool_ob=1)

    hacc = None
    for r in range(4):
        for q in range(4):
            part = jnp.dot(y[:, r, q, :], f1[r * 4 + q],
                           preferred_element_type=jnp.float32)
            hacc = part if hacc is None else hacc + part
    h = jnp.maximum(hacc + g1[...], 0.0).astype(bf16)
    o_ref[...] = jnp.dot(h, f2[...], preferred_element_type=jnp.float32) + g2[...]


def kernel(x_nchw, conv0_w, conv0_b, conv1_w, conv1_b, conv2_w, conv2_b,
           conv3_w, conv3_b, conv4_w, conv4_b, conv5_w, conv5_b,
           fc1_w, fc1_b, fc2_w, fc2_b):
    N = x_nchw.shape[0]
    H = W = 32
    bf16 = jnp.bfloat16

    xb = x_nchw[:, 0].astype(bf16)
    A = _conv0_matrix(conv0_w, W).astype(bf16)
    c0b = jnp.tile(conv0_b, 8).reshape(1, 256)

    ws = []
    for w, b, bw in ((conv1_w, conv1_b, 4), (conv2_w, conv2_b, 4),
                     (conv3_w, conv3_b, 4), (conv4_w, conv4_b, 2),
                     (conv5_w, conv5_b, 2)):
        cout = w.shape[3]
        ws.append((_blocked_matrix(w, bw).astype(bf16),
                   jnp.tile(b, bw).reshape(1, bw * cout)))
    f1 = fc1_w.reshape(16, 128, fc1_w.shape[1]).astype(bf16)
    f2 = fc2_w.astype(bf16)
    K = fc2_w.shape[1]

    T = _tile(N, 32)
    const2 = lambda n: (0, 0)
    const3 = lambda n: (0, 0, 0)
    in_specs = [
        pl.BlockSpec((T, H, W), lambda n: (n, 0, 0)),
        pl.BlockSpec(A.shape, const2),
        pl.BlockSpec(c0b.shape, const2),
    ]
    args = [xb, A, c0b]
    for wm, bm in ws:
        in_specs.append(pl.BlockSpec(wm.shape, const2))
        in_specs.append(pl.BlockSpec(bm.shape, const2))
        args.append(wm)
        args.append(bm)
    in_specs += [
        pl.BlockSpec(f1.shape, const3),
        pl.BlockSpec(fc1_b.shape, const2),
        pl.BlockSpec(f2.shape, const2),
        pl.BlockSpec(fc2_b.shape, const2),
    ]
    args += [f1, fc1_b, f2, fc2_b]

    out = pl.pallas_call(
        _fused_kernel,
        out_shape=jax.ShapeDtypeStruct((N, K), jnp.float32),
        grid=(N // T,),
        in_specs=in_specs,
        out_specs=pl.BlockSpec((T, K), lambda n: (n, 0)),
        compiler_params=pltpu.CompilerParams(
            dimension_semantics=("parallel",),
            vmem_limit_bytes=64 * 1024 * 1024),
    )(*args)
    return out


# speedup vs baseline: 4.8848x; 1.0307x over previous
import jax
import jax.numpy as jnp
from jax.experimental import pallas as pl
from jax.experimental.pallas import tpu as pltpu


def _tile(n, target):
    t = min(target, n)
    while n % t:
        t -= 1
    return t


def _conv0_matrix(w0, W):
    Cout = w0.shape[-1]
    w_in = jnp.arange(W)[:, None]
    w_out = jnp.arange(W)[None, :]
    dx = w_in - w_out + 1
    valid = (dx >= 0) & (dx < 3)
    tap = w0[:, :, 0, :]
    A = tap[:, jnp.clip(dx, 0, 2), :]
    A = jnp.where(valid[None, :, :, None], A, 0.0)
    return A.reshape(3 * W, W * Cout)


def _blocked_matrix(w, b):
    Cin, Cout = w.shape[2], w.shape[3]
    p = jnp.array(list(range(b)) + [-1, b])
    wb = jnp.arange(b)
    dx = p[:, None] - wb[None, :] + 1
    valid = (dx >= 0) & (dx < 3)
    A = w[:, jnp.clip(dx, 0, 2)]
    A = jnp.where(valid[None, :, :, None, None], A, 0.0)
    A = A.transpose(1, 3, 0, 2, 4)
    return A.reshape((b + 2) * Cin, 3 * b * Cout)


def _fused_kernel(x_ref, a0, c0b, w1, b1, w2, b2, w3, b3, w4, b4, w5, b5,
                  f1, g1, f2, g2, o_ref):
    T, H0, W0 = x_ref.shape
    bf16 = jnp.bfloat16

    zr = jnp.zeros((T, 1, W0), bf16)
    xpad = jnp.concatenate([zr, x_ref[...], zr], axis=1)
    rows3 = jnp.concatenate(
        [xpad[:, 0:H0, :], xpad[:, 1:H0 + 1, :], xpad[:, 2:H0 + 2, :]],
        axis=-1).reshape(T * H0, 3 * W0)
    pieces = []
    for q in range(0, W0 // 4, 2):
        d = jnp.dot(rows3, a0[:, q * 128:(q + 2) * 128],
                    preferred_element_type=jnp.float32)
        d = jnp.maximum(d + c0b[...], 0.0).astype(bf16)
        pieces.append(d[:, 0:128].reshape(T, H0, 1, 128))
        pieces.append(d[:, 128:256].reshape(T, H0, 1, 128))
    y = jnp.concatenate(pieces, axis=2)

    def conv(xb, w_ref, b_ref, bw, pool_ob=0):
        _, H, Q, C4 = xb.shape
        Cin = C4 // bw
        C4o = w_ref.shape[-1] // 3
        Cout = C4o // bw
        zrow = jnp.zeros((T, 1, Q, C4), bf16)
        src = jnp.concatenate([zrow, xb, zrow], axis=1)
        zq = jnp.zeros((T, H + 2, 1, Cin), bf16)
        pt = jnp.concatenate(
            [zq, src[:, :, 0:Q - 1, (bw - 1) * Cin:bw * Cin]], axis=2)
        nh = jnp.concatenate([src[:, :, 1:Q, 0:Cin], zq], axis=2)
        win = jnp.concatenate([src, pt, nh], axis=-1)
        part = jnp.dot(win.reshape(T * (H + 2) * Q, (bw + 2) * Cin), w_ref[...],
                       preferred_element_type=jnp.float32)
        p4 = part.reshape(T, H + 2, Q, 3 * C4o)
        acc = (p4[:, 0:H, :, 0:C4o]
               + p4[:, 1:H + 1, :, C4o:2 * C4o]
               + p4[:, 2:H + 2, :, 2 * C4o:3 * C4o])
        acc = jnp.maximum(acc + b_ref[...].reshape(1, 1, 1, C4o), 0.0)
        if not pool_ob:
            return acc.astype(bf16)
        H2 = H // 2
        x5 = acc.reshape(T, H2, 2, Q, C4o)
        v = jnp.maximum(x5[:, :, 0], x5[:, :, 1])
        hpieces = [jnp.maximum(v[..., (2 * j) * Cout:(2 * j + 1) * Cout],
                               v[..., (2 * j + 1) * Cout:(2 * j + 2) * Cout])
                   for j in range(bw // 2)]
        if pool_ob == bw // 2:
            return jnp.concatenate(hpieces, axis=-1).astype(bf16) \
                if len(hpieces) > 1 else hpieces[0].astype(bf16)
        vv = jnp.concatenate(hpieces, axis=-1)
        x6 = vv.reshape(T, H2, Q // 2, 2, (bw // 2) * Cout)
        return jnp.concatenate([x6[:, :, :, 0, :], x6[:, :, :, 1, :]],
                               axis=-1).astype(bf16)

    y = conv(y, w1, b1, bw=4, pool_ob=4)
    y = conv(y, w2, b2, bw=4)
    y = conv(y, w3, b3, bw=4, pool_ob=2)
    y = conv(y, w4, b4, bw=2)
    y = conv(y, w5, b5, bw=2, pool_ob=1)

    hacc = None
    for r in range(4):
        for q in range(4):
            part = jnp.dot(y[:, r, q, :], f1[r * 4 + q],
                           preferred_element_type=jnp.float32)
            hacc = part if hacc is None else hacc + part
    h = jnp.maximum(hacc + g1[...], 0.0).astype(bf16)
    o_ref[...] = jnp.dot(h, f2[...], preferred_element_type=jnp.float32) + g2[...]


def kernel(x_nchw, conv0_w, conv0_b, conv1_w, conv1_b, conv2_w, conv2_b,
           conv3_w, conv3_b, conv4_w, conv4_b, conv5_w, conv5_b,
           fc1_w, fc1_b, fc2_w, fc2_b):
    N = x_nchw.shape[0]
    H = W = 32
    bf16 = jnp.bfloat16

    xb = x_nchw[:, 0].astype(bf16)
    A = _conv0_matrix(conv0_w, W).astype(bf16)
    c0b = jnp.tile(conv0_b, 8).reshape(1, 256)

    ws = []
    for w, b, bw in ((conv1_w, conv1_b, 4), (conv2_w, conv2_b, 4),
                     (conv3_w, conv3_b, 4), (conv4_w, conv4_b, 2),
                     (conv5_w, conv5_b, 2)):
        cout = w.shape[3]
        ws.append((_blocked_matrix(w, bw).astype(bf16),
                   jnp.tile(b, bw).reshape(1, bw * cout)))
    f1 = fc1_w.reshape(16, 128, fc1_w.shape[1]).astype(bf16)
    f2 = fc2_w.astype(bf16)
    K = fc2_w.shape[1]

    T = _tile(N, 64)
    const2 = lambda n: (0, 0)
    const3 = lambda n: (0, 0, 0)
    in_specs = [
        pl.BlockSpec((T, H, W), lambda n: (n, 0, 0)),
        pl.BlockSpec(A.shape, const2),
        pl.BlockSpec(c0b.shape, const2),
    ]
    args = [xb, A, c0b]
    for wm, bm in ws:
        in_specs.append(pl.BlockSpec(wm.shape, const2))
        in_specs.append(pl.BlockSpec(bm.shape, const2))
        args.append(wm)
        args.append(bm)
    in_specs += [
        pl.BlockSpec(f1.shape, const3),
        pl.BlockSpec(fc1_b.shape, const2),
        pl.BlockSpec(f2.shape, const2),
        pl.BlockSpec(fc2_b.shape, const2),
    ]
    args += [f1, fc1_b, f2, fc2_b]

    out = pl.pallas_call(
        _fused_kernel,
        out_shape=jax.ShapeDtypeStruct((N, K), jnp.float32),
        grid=(N // T,),
        in_specs=in_specs,
        out_specs=pl.BlockSpec((T, K), lambda n: (n, 0)),
        compiler_params=pltpu.CompilerParams(
            dimension_semantics=("parallel",),
            vmem_limit_bytes=64 * 1024 * 1024),
    )(*args)
    return out
